# Initial kernel scaffold; baseline (speedup 1.0000x reference)
import jax
import jax.numpy as jnp
from jax.experimental import pallas as pl


def kernel(x):
    raise NotImplementedError("write your pallas kernel here")


# speedup vs baseline: 8.2951x; 8.2951x over previous
import functools

import jax
import jax.numpy as jnp
from jax import lax
from jax.experimental import pallas as pl
from jax.experimental.pallas import tpu as pltpu
from jax.experimental.pallas import tpu_sc as plsc

B, N = 64, 8192
L = 16
K = 8
CHUNK = L * K
NSTEP = N // CHUNK
NC, NS = 2, 16
NW = NC * NS
RPW = B // NW
NBIS = 24
NMICH = 2

_f32 = jnp.float32


def _allmax(v):
    for s in (8, 4, 2, 1):
        idx = lax.iota(jnp.int32, L) ^ s
        v = jnp.maximum(v, jnp.take(v, idx))
    return v


def _allsum(v):
    for s in (8, 4, 2, 1):
        idx = lax.iota(jnp.int32, L) ^ s
        v = v + jnp.take(v, idx)
    return v


def _sc_body(x_hbm, out_hbm, x_v, out_v):
    wid = lax.axis_index("s") * NC + lax.axis_index("c")
    base = wid * RPW
    pltpu.sync_copy(x_hbm.at[pl.ds(base, RPW)], x_v)

    for r in range(RPW):
        def ms_body(j, carry):
            ms, ss = carry
            b0 = j * CHUNK
            ms2, ss2 = [], []
            for u in range(K):
                v = x_v[r, pl.ds(b0 + u * L, L)]
                ms2.append(jnp.maximum(ms[u], v))
                ss2.append(ss[u] + v)
            return tuple(ms2), tuple(ss2)

        init = (
            tuple(jnp.full((L,), -3.0e38, _f32) for _ in range(K)),
            tuple(jnp.zeros((L,), _f32) for _ in range(K)),
        )
        ms, ss = lax.fori_loop(0, NSTEP, ms_body, init)
        vm, vs = ms[0], ss[0]
        for u in range(1, K):
            vm = jnp.maximum(vm, ms[u])
            vs = vs + ss[u]
        row_max = _allmax(vm)
        row_sum = _allsum(vs)

        lo = jnp.maximum(row_max - 1.0, (row_sum - 1.0) * (1.0 / N))
        hi = row_max

        def bis_body(t, lh):
            blo, bhi = lh
            mid = 0.5 * (blo + bhi)

            def s_body(j, accs):
                b0 = j * CHUNK
                out = []
                for u in range(K):
                    v = x_v[r, pl.ds(b0 + u * L, L)]
                    out.append(accs[u] + jnp.maximum(v - mid, 0.0))
                return tuple(out)

            accs = lax.fori_loop(
                0, NSTEP, s_body, tuple(jnp.zeros((L,), _f32) for _ in range(K))
            )
            va = accs[0]
            for u in range(1, K):
                va = va + accs[u]
            s_mid = _allsum(va)
            pred = s_mid >= 1.0
            return jnp.where(pred, mid, blo), jnp.where(pred, bhi, mid)

        lo, hi = lax.fori_loop(0, NBIS, bis_body, (lo, hi))

        def mich_body(t, tau):
            def inner(j, carry):
                sa, ca = carry
                b0 = j * CHUNK
                sa2, ca2 = [], []
                for u in range(K):
                    v = x_v[r, pl.ds(b0 + u * L, L)]
                    m = v > tau
                    sa2.append(sa[u] + jnp.where(m, v, 0.0))
                    ca2.append(ca[u] + jnp.where(m, 1.0, 0.0))
                return tuple(sa2), tuple(ca2)

            z = tuple(jnp.zeros((L,), _f32) for _ in range(K))
            sa, ca = lax.fori_loop(0, NSTEP, inner, (z, z))
            vsa, vca = sa[0], ca[0]
            for u in range(1, K):
                vsa = vsa + sa[u]
                vca = vca + ca[u]
            s_tot = _allsum(vsa)
            c_tot = _allsum(vca)
            return (s_tot - 1.0) / c_tot

        tau = lax.fori_loop(0, NMICH, mich_body, lo)

        def out_body(j, _):
            b0 = j * CHUNK
            for u in range(K):
                v = x_v[r, pl.ds(b0 + u * L, L)]
                out_v[r, pl.ds(b0 + u * L, L)] = jnp.maximum(v - tau, 0.0)
            return 0

        lax.fori_loop(0, NSTEP, out_body, 0)

    pltpu.sync_copy(out_v, out_hbm.at[pl.ds(base, RPW)])


_sparsemax_sc = functools.partial(
    pl.kernel,
    mesh=plsc.VectorSubcoreMesh(core_axis_name="c", subcore_axis_name="s"),
    out_type=jax.ShapeDtypeStruct((B, N), _f32),
    scratch_types=[
        pltpu.VMEM((RPW, N), _f32),
        pltpu.VMEM((RPW, N), _f32),
    ],
)(_sc_body)


@jax.jit
def kernel(x):
    return _sparsemax_sc(x)
